# Initial kernel scaffold; baseline (speedup 1.0000x reference)
import jax
import jax.numpy as jnp
from jax.experimental import pallas as pl


def kernel(h, edge_index, r, edge_attr, basis, params):
    raise NotImplementedError("write your pallas kernel here")


# speedup vs baseline: 9.7505x; 9.7505x over previous
import functools
import math

import jax
import jax.numpy as jnp
from jax import lax
from jax.experimental import pallas as pl
from jax.experimental.pallas import tpu as pltpu
from jax.experimental.pallas import tpu_sc as plsc

N = 10000
E = 320000
D = 128
HD = 64
H = 4
DH = 32
RH = 32
ED_ = 16

NC = 2
NS = 16
EPT = E // NS
RPT = 624
TAIL = N - NS * RPT
CE = 80
NCH = EPT // CE
DEN_W = 16

_f32 = jnp.float32


_NB = 10
_BN = N // _NB
_EB = 2000


def _row_spec(width):
    return pl.BlockSpec((_BN, width), lambda i: (i, 0))


def _full_spec(shape):
    return pl.BlockSpec(shape, lambda i: tuple(0 for _ in shape))


def _split_spec(width):
    return pl.BlockSpec((NC, _BN, width), lambda i: (0, i, 0))


def _qkv_body(x_ref, wq_ref, wk_ref, wv_ref, q_ref, k_ref, v_ref):
    x = x_ref[...]
    q = jnp.dot(x, wq_ref[...], preferred_element_type=_f32)
    k = jnp.dot(x, wk_ref[...], preferred_element_type=_f32)
    v = jnp.dot(x, wv_ref[...], preferred_element_type=_f32)
    q_ref[0], q_ref[1] = q[:, :HD], q[:, HD:]
    k_ref[0], k_ref[1] = k[:, :HD], k[:, HD:]
    v_ref[0], v_ref[1] = v[:, :HD], v[:, HD:]


def _qkv(x, wq, wk, wv):
    outs = pl.pallas_call(
        _qkv_body,
        grid=(_NB,),
        in_specs=[_row_spec(D), _full_spec((D, D)), _full_spec((D, D)),
                  _full_spec((D, D))],
        out_specs=[_split_spec(HD)] * 3,
        out_shape=[jax.ShapeDtypeStruct((NC, N, HD), _f32)] * 3,
    )(x, wq, wk, wv)
    return tuple(o.reshape(NC * N, HD) for o in outs)


def _lin_body(x_ref, w_ref, o_ref):
    o = jnp.dot(x_ref[...], w_ref[...], preferred_element_type=_f32)
    o_ref[0], o_ref[1] = o[:, :HD], o[:, HD:]


def _lin_split(x, w):
    out = pl.pallas_call(
        _lin_body,
        grid=(_NB,),
        in_specs=[_row_spec(D), _full_spec((D, D))],
        out_specs=_split_spec(HD),
        out_shape=jax.ShapeDtypeStruct((NC, N, HD), _f32),
    )(x, w)
    return out.reshape(NC * N, HD)


def _edge_mlp_body(r_ref, ea_ref, bs_ref,
                   w1ra, w1ea, b1a, w2a, b2a,
                   w1rb, w1eb, b1b, w2b, b2b,
                   w1rc, w1ec, b1c, w2c, b2c,
                   o1_ref, o2_ref, oc_ref):
    r = r_ref[...]
    ea = ea_ref[...]
    bs = bs_ref[...]

    def mlp(w1r, w1e, b1, w2, b2):
        hid = jnp.dot(r, w1r[...], preferred_element_type=_f32)
        hid = hid + jnp.dot(ea, w1e[...], preferred_element_type=_f32)
        hid = jnp.maximum(hid + b1[...], 0.0)
        return (jnp.dot(hid, w2[...], preferred_element_type=_f32) + b2[...]) * bs

    r1 = mlp(w1ra, w1ea, b1a, w2a, b2a)
    o1_ref[0], o1_ref[1] = r1[:, :D], r1[:, D:]
    r2 = mlp(w1rb, w1eb, b1b, w2b, b2b)
    o2_ref[0], o2_ref[1] = r2[:, :D], r2[:, D:]
    rc = mlp(w1rc, w1ec, b1c, w2c, b2c)
    oc_ref[0], oc_ref[1] = rc[:, :HD], rc[:, HD:]


def _perm_att_w2(w2, b2):
    cols = jnp.concatenate(
        [w2[:, :HD], w2[:, D:D + HD], w2[:, HD:D], w2[:, D + HD:]], axis=1)
    bias = jnp.concatenate(
        [b2[:HD], b2[D:D + HD], b2[HD:D], b2[D + HD:]]).reshape(1, 2 * D)
    return cols, bias


def _edge_mlp(r, ea, bs, p):
    espec = lambda w: pl.BlockSpec((_EB, w), lambda i: (i, 0))
    osplit = lambda w: pl.BlockSpec((NC, _EB, w), lambda i: (0, i, 0))
    args = [r, ea, bs]
    in_specs = [espec(1), espec(ED_), espec(1)]
    for pre in ('att1', 'att2'):
        w1 = p[pre + '_Wr1']
        w2, b2 = _perm_att_w2(p[pre + '_Wr2'], p[pre + '_br2'])
        args += [w1[:1], w1[1:], p[pre + '_br1'].reshape(1, RH), w2, b2]
        in_specs += [_full_spec((1, RH)), _full_spec((ED_, RH)),
                     _full_spec((1, RH)), _full_spec((RH, 2 * D)),
                     _full_spec((1, 2 * D))]
    w1 = p['conv_Wr1']
    args += [w1[:1], w1[1:], p['conv_br1'].reshape(1, RH),
             p['conv_Wr2'], p['conv_br2'].reshape(1, D)]
    in_specs += [_full_spec((1, RH)), _full_spec((ED_, RH)),
                 _full_spec((1, RH)), _full_spec((RH, D)),
                 _full_spec((1, D))]
    o1, o2, oc = pl.pallas_call(
        _edge_mlp_body,
        grid=(E // _EB,),
        in_specs=in_specs,
        out_specs=[osplit(D), osplit(D), osplit(HD)],
        out_shape=[jax.ShapeDtypeStruct((NC, E, D), _f32),
                   jax.ShapeDtypeStruct((NC, E, D), _f32),
                   jax.ShapeDtypeStruct((NC, E, HD), _f32)],
    )(*args)
    return (o1.reshape(NC * E, D), o2.reshape(NC * E, D),
            oc.reshape(NC * E, HD))


def _post_att_body(x_ref, np_ref, dp_ref, exp_ref, woa_ref, woh_ref, bo_ref,
                   g_ref, b_ref, o_ref):
    num = jnp.concatenate([np_ref[0], np_ref[1]], axis=-1)
    den4 = jnp.concatenate([dp_ref[0][:, :2], dp_ref[1][:, :2]], axis=-1)
    denb = jnp.dot(den4, exp_ref[...], preferred_element_type=_f32)
    agg = num / (denb + 1e-9)
    x = x_ref[...]
    out = (jnp.dot(agg, woa_ref[...], preferred_element_type=_f32)
           + jnp.dot(x, woh_ref[...], preferred_element_type=_f32)
           + bo_ref[...])
    nrm = jnp.abs(out)
    phase = out / (nrm + 1e-6)
    mu = jnp.mean(nrm, axis=-1, keepdims=True)
    sd = jnp.sqrt(jnp.mean((nrm - mu) ** 2, axis=-1, keepdims=True))
    ln = (nrm - mu) / (sd + 1e-6) * g_ref[...] + b_ref[...]
    o_ref[...] = jnp.maximum(ln, 0.0) * phase


def _post_att(x, npart, dpart, wo, bo, g, b):
    expand = jnp.repeat(jnp.eye(H, dtype=_f32), DH, axis=1)
    return pl.pallas_call(
        _post_att_body,
        grid=(_NB,),
        in_specs=[_row_spec(D), _split_spec(HD), _split_spec(DEN_W),
                  _full_spec((H, D)), _full_spec((D, D)), _full_spec((D, D)),
                  _full_spec((1, D)), _full_spec((1, D)), _full_spec((1, D))],
        out_specs=_row_spec(D),
        out_shape=jax.ShapeDtypeStruct((N, D), _f32),
    )(x, npart, dpart, expand, wo[:D], wo[D:], bo.reshape(1, D),
      g.reshape(1, D), b.reshape(1, D))


def _post_conv_body(x_ref, np_ref, dp_ref, wself_ref, bself_ref, o_ref):
    num = jnp.concatenate([np_ref[0], np_ref[1]], axis=-1)
    deg = dp_ref[0][:, :1]
    agg = num / jnp.maximum(deg, 1.0)
    o_ref[...] = (agg
                  + jnp.dot(x_ref[...], wself_ref[...],
                            preferred_element_type=_f32)
                  + bself_ref[...])


def _post_conv(x, npart, dpart, wself, bself):
    return pl.pallas_call(
        _post_conv_body,
        grid=(_NB,),
        in_specs=[_row_spec(D), _split_spec(HD), _split_spec(DEN_W),
                  _full_spec((D, D)), _full_spec((1, D))],
        out_specs=_row_spec(D),
        out_shape=jax.ShapeDtypeStruct((N, D), _f32),
    )(x, npart, dpart, wself, bself.reshape(1, D))


def _tab_zero(s, zn_hbm, zd_hbm, num_tab, den_tab):
    row0 = s * RPT
    pltpu.sync_copy(zn_hbm, num_tab.at[pl.ds(row0, RPT)])
    pltpu.sync_copy(zd_hbm, den_tab.at[pl.ds(row0, RPT)])

    @pl.when(s == 0)
    def _():
        pltpu.sync_copy(zn_hbm.at[pl.ds(0, TAIL)],
                        num_tab.at[pl.ds(NS * RPT, TAIL)])
        pltpu.sync_copy(zd_hbm.at[pl.ds(0, TAIL)],
                        den_tab.at[pl.ds(NS * RPT, TAIL)])


def _tab_out(c, s, num_tab, den_tab, num_out, den_out):
    row0 = s * RPT
    pltpu.sync_copy(num_tab.at[pl.ds(row0, RPT)],
                    num_out.at[c, pl.ds(row0, RPT)])
    pltpu.sync_copy(den_tab.at[pl.ds(row0, RPT)],
                    den_out.at[c, pl.ds(row0, RPT)])

    @pl.when(s == 0)
    def _():
        pltpu.sync_copy(num_tab.at[pl.ds(NS * RPT, TAIL)],
                        num_out.at[c, pl.ds(NS * RPT, TAIL)])
        pltpu.sync_copy(den_tab.at[pl.ds(NS * RPT, TAIL)],
                        den_out.at[c, pl.ds(NS * RPT, TAIL)])


def _shift_idx(idx_ref, out_ref, off):
    for o in range(0, CE, 16):
        out_ref[pl.ds(o, 16)] = idx_ref[pl.ds(o, 16)] + off


def _sc_att_body(q_hbm, k_hbm, v_hbm, rad_hbm, src_hbm, dst_hbm,
                 zn_hbm, zd_hbm, num_out, den_out,
                 src_v, dst_v, dst2_v, k_rows, v_rows, q_rows, rad_rows,
                 den_buf, num_tab, den_tab, sem1, sem2, sem3):
    c = lax.axis_index("c")
    s = lax.axis_index("s")
    _tab_zero(s, zn_hbm, zd_hbm, num_tab, den_tab)
    plsc.subcore_barrier()

    base = s * EPT
    roff = c * N
    eoff = c * E
    isq = 1.0 / math.sqrt(DH)
    lane = lax.iota(jnp.int32, 16)

    def chunk(j, carry):
        eb = base + j * CE
        pltpu.sync_copy(src_hbm.at[pl.ds(eb, CE)], src_v)
        pltpu.sync_copy(dst_hbm.at[pl.ds(eb, CE)], dst_v)
        _shift_idx(src_v, src_v, roff)
        _shift_idx(dst_v, dst2_v, roff)
        cp1 = pltpu.async_copy(k_hbm.at[src_v], k_rows, sem1)
        cp2 = pltpu.async_copy(v_hbm.at[src_v], v_rows, sem2)
        cp3 = pltpu.async_copy(q_hbm.at[dst2_v], q_rows, sem3)
        pltpu.sync_copy(rad_hbm.at[pl.ds(eoff + eb, CE)], rad_rows)
        cp1.wait()
        cp2.wait()
        cp3.wait()

        def edge(i, carry2):
            dv = jnp.zeros((16,), _f32)
            for hh in range(2):
                c0 = hh * DH
                t = (q_rows[i, pl.ds(c0, 16)]
                     * k_rows[i, pl.ds(c0, 16)]
                     * rad_rows[i, pl.ds(c0, 16)])
                t = t + (q_rows[i, pl.ds(c0 + 16, 16)]
                         * k_rows[i, pl.ds(c0 + 16, 16)]
                         * rad_rows[i, pl.ds(c0 + 16, 16)])
                ev = jnp.exp(jnp.full((16,), jnp.sum(t) * isq, _f32))
                for o in (c0, c0 + 16):
                    v_rows[i, pl.ds(o, 16)] = (
                        v_rows[i, pl.ds(o, 16)]
                        * rad_rows[i, pl.ds(HD + o, 16)] * ev)
                dv = jnp.where(lane == hh, ev, dv)
            den_buf[i, pl.ds(0, DEN_W)] = dv
            return carry2

        lax.fori_loop(0, CE, edge, 0)
        pltpu.sync_copy(v_rows, num_tab.at[dst_v], add=True)
        pltpu.sync_copy(den_buf, den_tab.at[dst_v], add=True)
        return carry

    lax.fori_loop(0, NCH, chunk, 0)
    plsc.subcore_barrier()
    _tab_out(c, s, num_tab, den_tab, num_out, den_out)


def _sc_att(q2, k2, v2, rad2, src, dst, zn, zd):
    mesh = plsc.VectorSubcoreMesh(core_axis_name="c", subcore_axis_name="s")
    f = pl.kernel(
        _sc_att_body,
        compiler_params=pltpu.CompilerParams(needs_layout_passes=False, use_tc_tiling_on_sc=False),
        out_type=[jax.ShapeDtypeStruct((NC, N, HD), _f32),
                  jax.ShapeDtypeStruct((NC, N, DEN_W), _f32)],
        mesh=mesh,
        scratch_types=[
            pltpu.VMEM((CE,), jnp.int32),
            pltpu.VMEM((CE,), jnp.int32),
            pltpu.VMEM((CE,), jnp.int32),
            pltpu.VMEM((CE, HD), _f32),
            pltpu.VMEM((CE, HD), _f32),
            pltpu.VMEM((CE, HD), _f32),
            pltpu.VMEM((CE, 2 * HD), _f32),
            pltpu.VMEM((CE, DEN_W), _f32),
            pltpu.VMEM_SHARED((N, HD), _f32),
            pltpu.VMEM_SHARED((N, DEN_W), _f32),
            pltpu.SemaphoreType.DMA,
            pltpu.SemaphoreType.DMA,
            pltpu.SemaphoreType.DMA,
        ],
    )
    return f(q2, k2, v2, rad2, src, dst, zn, zd)


def _sc_conv_body(x_hbm, rad_hbm, src_hbm, dst_hbm, zn_hbm, zd_hbm,
                  num_out, den_out,
                  src_v, dst_v, g_rows, rad_rows, den_buf,
                  num_tab, den_tab, sem1):
    c = lax.axis_index("c")
    s = lax.axis_index("s")
    _tab_zero(s, zn_hbm, zd_hbm, num_tab, den_tab)
    plsc.subcore_barrier()

    base = s * EPT
    roff = c * N
    eoff = c * E
    lane = lax.iota(jnp.int32, 16)
    one0 = jnp.where(lane == 0, 1.0, 0.0).astype(_f32)

    def fill(i, carry):
        den_buf[i, pl.ds(0, DEN_W)] = one0
        return carry

    lax.fori_loop(0, CE, fill, 0)

    def chunk(j, carry):
        eb = base + j * CE
        pltpu.sync_copy(src_hbm.at[pl.ds(eb, CE)], src_v)
        pltpu.sync_copy(dst_hbm.at[pl.ds(eb, CE)], dst_v)
        _shift_idx(src_v, src_v, roff)
        cp1 = pltpu.async_copy(x_hbm.at[src_v], g_rows, sem1)
        pltpu.sync_copy(rad_hbm.at[pl.ds(eoff + eb, CE)], rad_rows)
        cp1.wait()

        def edge(i, carry2):
            for jj in range(HD // 16):
                o = jj * 16
                g_rows[i, pl.ds(o, 16)] = (g_rows[i, pl.ds(o, 16)]
                                           * rad_rows[i, pl.ds(o, 16)])
            return carry2

        lax.fori_loop(0, CE, edge, 0)
        pltpu.sync_copy(g_rows, num_tab.at[dst_v], add=True)
        pltpu.sync_copy(den_buf, den_tab.at[dst_v], add=True)
        return carry

    lax.fori_loop(0, NCH, chunk, 0)
    plsc.subcore_barrier()
    _tab_out(c, s, num_tab, den_tab, num_out, den_out)


def _sc_conv(x2, radc2, src, dst, zn, zd):
    mesh = plsc.VectorSubcoreMesh(core_axis_name="c", subcore_axis_name="s")
    f = pl.kernel(
        _sc_conv_body,
        compiler_params=pltpu.CompilerParams(needs_layout_passes=False, use_tc_tiling_on_sc=False),
        out_type=[jax.ShapeDtypeStruct((NC, N, HD), _f32),
                  jax.ShapeDtypeStruct((NC, N, DEN_W), _f32)],
        mesh=mesh,
        scratch_types=[
            pltpu.VMEM((CE,), jnp.int32),
            pltpu.VMEM((CE,), jnp.int32),
            pltpu.VMEM((CE, HD), _f32),
            pltpu.VMEM((CE, HD), _f32),
            pltpu.VMEM((CE, DEN_W), _f32),
            pltpu.VMEM_SHARED((N, HD), _f32),
            pltpu.VMEM_SHARED((N, DEN_W), _f32),
            pltpu.SemaphoreType.DMA,
        ],
    )
    return f(x2, radc2, src, dst, zn, zd)


def kernel(h, edge_index, r, edge_attr, basis, params):
    p = params
    src = edge_index[0].astype(jnp.int32)
    dst = edge_index[1].astype(jnp.int32)
    zn = jnp.zeros((RPT, HD), _f32)
    zd = jnp.zeros((RPT, DEN_W), _f32)

    rad1, rad2, radc = _edge_mlp(r, edge_attr, basis, p)

    q, k, v = _qkv(h, p['att1_Wq'], p['att1_Wk'], p['att1_Wv'])
    npart, dpart = _sc_att(q, k, v, rad1, src, dst, zn, zd)
    x = _post_att(h, npart, dpart, p['att1_Wo'], p['att1_bo'],
                  p['norm1_g'], p['norm1_b'])

    q, k, v = _qkv(x, p['att2_Wq'], p['att2_Wk'], p['att2_Wv'])
    npart, dpart = _sc_att(q, k, v, rad2, src, dst, zn, zd)
    x = _post_att(x, npart, dpart, p['att2_Wo'], p['att2_bo'],
                  p['norm2_g'], p['norm2_b'])

    xc = _lin_split(x, p['conv_W'])
    npart, dpart = _sc_conv(xc, radc, src, dst, zn, zd)
    return _post_conv(x, npart, dpart, p['conv_Wself'], p['conv_bself'])
